# baseline (speedup 1.0000x reference)
import jax
import jax.numpy as jnp
from jax import lax
from jax.experimental import pallas as pl
from jax.experimental.pallas import tpu as pltpu
from jax.experimental.pallas import tpu_sc as plsc

_T = 4096
_B = 16
_C = 256
_NW = 32
_ROWS_W = _T // _NW
_CH = 2
_NCH = _ROWS_W // _CH
_NG = _NCH // 4
_LANES = 16
_CPB = _C // _LANES
_R = 1.0 / (2 * _T - 1)


def _body(y_hbm, out_hbm, in_v, out_v, sin, sout):
    c = lax.axis_index("c")
    s = lax.axis_index("s")
    wid = s * 2 + c
    base = wid * _ROWS_W

    def issue_chunk(ci, b):
        pltpu.async_copy(y_hbm.at[pl.ds(base + _CH * ci, _CH)], in_v[b],
                         sin[b])

    def drain_chunk(b):
        pltpu.make_async_copy(y_hbm.at[pl.ds(0, _CH)], in_v[b],
                              sin[b]).wait()

    def issue_tail_halo():
        pltpu.async_copy(
            y_hbm.at[pl.ds(jnp.minimum(base + _ROWS_W, _T - 1), 1)],
            in_v[0].at[pl.ds(0, 1)], sin[0])

    def drain_tail_halo():
        pltpu.make_async_copy(y_hbm.at[pl.ds(0, 1)],
                              in_v[0].at[pl.ds(0, 1)], sin[0]).wait()

    def wait_out(ob):
        pltpu.make_async_copy(out_v[ob], out_hbm.at[pl.ds(0, 4 * _CH)],
                              sout[ob]).wait()

    def compute(ci, b):
        prev, cur, nxt = in_v[(b - 1) % 4], in_v[b], in_v[(b + 1) % 4]
        ov = out_v[b // 2]
        off = (b % 2) * 2 * _CH
        m0 = (base + _CH * ci).astype(jnp.float32)
        a0 = m0 * _R
        b0 = (float(_T - 1) - m0) * _R
        a1 = (m0 + 1.0) * _R
        b1 = (float(_T - 2) - m0) * _R

        @plsc.parallel_loop(0, _B, 1)
        def subloop(sub):
            for k in range(_CPB):
                sl = pl.ds(k * _LANES, _LANES)
                ym1 = prev[1, sub, sl]
                y0 = cur[0, sub, sl]
                y1 = cur[1, sub, sl]
                y2 = nxt[0, sub, sl]
                d0 = y0 - ym1
                d1 = y1 - y0
                d2 = y2 - y1
                ov[off + 0, sub, sl] = y0 - a0 * d0
                ov[off + 1, sub, sl] = y0 + b0 * d1
                ov[off + 2, sub, sl] = y1 - a1 * d1
                ov[off + 3, sub, sl] = y1 + b1 * d2

    pltpu.async_copy(y_hbm.at[pl.ds(jnp.maximum(base - 1, 0), 1)],
                     in_v[3].at[pl.ds(1, 1)], sin[3])
    issue_chunk(0, 0)
    issue_chunk(1, 1)
    issue_chunk(2, 2)
    pltpu.make_async_copy(y_hbm.at[pl.ds(0, 1)],
                          in_v[3].at[pl.ds(1, 1)], sin[3]).wait()
    drain_chunk(0)

    @pl.loop(0, _NG)
    def g_loop(g):
        for b in range(4):
            ci = 4 * g + b

            if b < 3:
                drain_chunk(b + 1)
            else:
                @pl.when(g < _NG - 1)
                def _():
                    drain_chunk(0)

                @pl.when(g == _NG - 1)
                def _():
                    drain_tail_halo()

            if b in (0, 2):
                @pl.when(g > 0)
                def _():
                    wait_out(b // 2)

            compute(ci, b)
            if b in (1, 3):
                pltpu.async_copy(
                    out_v[b // 2],
                    out_hbm.at[pl.ds(2 * (base + _CH * (ci - 1)), 4 * _CH)],
                    sout[b // 2])

            if b == 0:
                issue_chunk(ci + 3, 3)
            elif b == 1:
                @pl.when(g < _NG - 1)
                def _():
                    issue_chunk(ci + 3, 0)

                @pl.when(g == _NG - 1)
                def _():
                    issue_tail_halo()
            else:
                @pl.when(g < _NG - 1)
                def _():
                    issue_chunk(ci + 3, (b + 3) % 4)

    wait_out(0)
    wait_out(1)


@jax.jit
def kernel(y):
    T, B, C = y.shape
    call = pl.kernel(
        _body,
        out_type=jax.ShapeDtypeStruct((2 * T, B, C), jnp.float32),
        mesh=plsc.VectorSubcoreMesh(core_axis_name="c", subcore_axis_name="s"),
        scratch_types=[
            [pltpu.VMEM((_CH, _B, _C), jnp.float32) for _ in range(4)],
            [pltpu.VMEM((4 * _CH, _B, _C), jnp.float32) for _ in range(2)],
            [pltpu.SemaphoreType.DMA for _ in range(4)],
            [pltpu.SemaphoreType.DMA for _ in range(2)],
        ],
    )
    return call(y)


# speedup vs baseline: 1.0034x; 1.0034x over previous
import jax
import jax.numpy as jnp
from jax import lax
from jax.experimental import pallas as pl
from jax.experimental.pallas import tpu as pltpu
from jax.experimental.pallas import tpu_sc as plsc

_T = 4096
_B = 16
_C = 256
_NW = 32
_ROWS_W = _T // _NW
_CH = 2
_NCH = _ROWS_W // _CH
_NG = _NCH // 4
_LANES = 16
_CPB = _C // _LANES
_R = 1.0 / (2 * _T - 1)


def _body(y_hbm, out_hbm, in_v, out_v, sin, sout):
    c = lax.axis_index("c")
    s = lax.axis_index("s")
    wid = s * 2 + c
    base = wid * _ROWS_W

    def issue_chunk(ci, b):
        pltpu.async_copy(y_hbm.at[pl.ds(base + _CH * ci, _CH)], in_v[b],
                         sin[b])

    def drain_chunk(b):
        pltpu.make_async_copy(y_hbm.at[pl.ds(0, _CH)], in_v[b],
                              sin[b]).wait()

    def issue_tail_halo():
        pltpu.async_copy(
            y_hbm.at[pl.ds(jnp.minimum(base + _ROWS_W, _T - 1), 1)],
            in_v[0].at[pl.ds(0, 1)], sin[0])

    def drain_tail_halo():
        pltpu.make_async_copy(y_hbm.at[pl.ds(0, 1)],
                              in_v[0].at[pl.ds(0, 1)], sin[0]).wait()

    def wait_out(ob):
        pltpu.make_async_copy(out_v[ob], out_hbm.at[pl.ds(0, 2 * _CH)],
                              sout[ob]).wait()

    def compute(ci, b):
        prev, cur, nxt = in_v[(b - 1) % 4], in_v[b], in_v[(b + 1) % 4]
        ov = out_v[b % 2]
        m0 = (base + _CH * ci).astype(jnp.float32)
        a0 = m0 * _R
        b0 = (float(_T - 1) - m0) * _R
        a1 = (m0 + 1.0) * _R
        b1 = (float(_T - 2) - m0) * _R

        @plsc.parallel_loop(0, _B, 1)
        def subloop(sub):
            for k in range(_CPB):
                sl = pl.ds(k * _LANES, _LANES)
                ym1 = prev[1, sub, sl]
                y0 = cur[0, sub, sl]
                y1 = cur[1, sub, sl]
                y2 = nxt[0, sub, sl]
                d0 = y0 - ym1
                d1 = y1 - y0
                d2 = y2 - y1
                ov[0, sub, sl] = y0 - a0 * d0
                ov[1, sub, sl] = y0 + b0 * d1
                ov[2, sub, sl] = y1 - a1 * d1
                ov[3, sub, sl] = y1 + b1 * d2

    pltpu.async_copy(y_hbm.at[pl.ds(jnp.maximum(base - 1, 0), 1)],
                     in_v[3].at[pl.ds(1, 1)], sin[3])
    issue_chunk(0, 0)
    issue_chunk(1, 1)
    issue_chunk(2, 2)
    pltpu.make_async_copy(y_hbm.at[pl.ds(0, 1)],
                          in_v[3].at[pl.ds(1, 1)], sin[3]).wait()
    drain_chunk(0)

    @pl.loop(0, _NG)
    def g_loop(g):
        for b in range(4):
            ci = 4 * g + b

            if b < 3:
                drain_chunk(b + 1)
            else:
                @pl.when(g < _NG - 1)
                def _():
                    drain_chunk(0)

                @pl.when(g == _NG - 1)
                def _():
                    drain_tail_halo()

            if b < 2:
                @pl.when(g > 0)
                def _():
                    wait_out(b % 2)
            else:
                wait_out(b % 2)

            compute(ci, b)
            pltpu.async_copy(
                out_v[b % 2],
                out_hbm.at[pl.ds(2 * (base + _CH * ci), 2 * _CH)],
                sout[b % 2])

            if b == 0:
                issue_chunk(ci + 3, 3)
            elif b == 1:
                @pl.when(g < _NG - 1)
                def _():
                    issue_chunk(ci + 3, 0)

                @pl.when(g == _NG - 1)
                def _():
                    issue_tail_halo()
            else:
                @pl.when(g < _NG - 1)
                def _():
                    issue_chunk(ci + 3, (b + 3) % 4)

    wait_out(0)
    wait_out(1)


@jax.jit
def kernel(y):
    T, B, C = y.shape
    call = pl.kernel(
        _body,
        out_type=jax.ShapeDtypeStruct((2 * T, B, C), jnp.float32),
        mesh=plsc.VectorSubcoreMesh(core_axis_name="c", subcore_axis_name="s"),
        scratch_types=[
            [pltpu.VMEM((_CH, _B, _C), jnp.float32) for _ in range(4)],
            [pltpu.VMEM((2 * _CH, _B, _C), jnp.float32) for _ in range(2)],
            [pltpu.SemaphoreType.DMA for _ in range(4)],
            [pltpu.SemaphoreType.DMA for _ in range(2)],
        ],
    )
    return call(y)
